# baseline (speedup 1.0000x reference)
import functools

import jax
import jax.numpy as jnp
from jax import lax
from jax.experimental import pallas as pl
from jax.experimental.pallas import tpu as pltpu
from jax.experimental.pallas import tpu_sc as plsc

N = 10000
E = 320000
SF = 9
DF = 4
H = 128
T = 3

NC = 2
NS = 16
NP = 10240
ROWS = NP // NS
CH = 128
PADROW = NP - 8

K = 4

W1 = NC * NS
E1C = -(-(E // W1) // (CH * K)) * K
E1 = W1 * E1C * CH

_f32 = jnp.float32
_i32 = jnp.int32


def _stage1_body(s_hbm, z_hbm, src_hbm, dst_hbm, out_hbm, srcv, dstv, gbuf,
                 semg, sema, acc):
    cid = lax.axis_index("c")
    sid = lax.axis_index("s")
    wid = cid * NS + sid
    rows = pl.ds(sid * ROWS, ROWS)

    pltpu.sync_copy(z_hbm.at[rows, :], acc.at[rows, :])
    pltpu.sync_copy(src_hbm.at[wid], srcv)
    pltpu.sync_copy(dst_hbm.at[wid], dstv)
    plsc.subcore_barrier()

    def body(p, c):
        j = p * K
        hg = [pltpu.async_copy(s_hbm.at[srcv.at[j + b]], gbuf.at[b], semg)
              for b in range(K)]
        for h in hg:
            h.wait()
        ha = [pltpu.async_copy(gbuf.at[b], acc.at[dstv.at[j + b]], sema,
                               add=True)
              for b in range(K)]
        for h in ha:
            h.wait()
        return c
    lax.fori_loop(0, E1C // K, body, 0)
    plsc.subcore_barrier()

    pltpu.sync_copy(acc.at[rows, :], out_hbm.at[cid, rows, :])


@functools.cache
def _get_stage1():
    return functools.partial(
        pl.kernel,
        out_type=jax.ShapeDtypeStruct((NC, NP, 32), _f32),
        mesh=plsc.VectorSubcoreMesh(core_axis_name="c", subcore_axis_name="s",
                                    num_cores=NC, num_subcores=NS),
        compiler_params=pltpu.CompilerParams(use_tc_tiling_on_sc=False),
        scratch_types=[
            pltpu.VMEM((E1C, CH), _i32),
            pltpu.VMEM((E1C, CH), _i32),
            pltpu.VMEM((K, CH, 32), _f32),
            pltpu.SemaphoreType.DMA,
            pltpu.SemaphoreType.DMA,
            pltpu.VMEM_SHARED((NP, 32), _f32),
        ],
    )(_stage1_body)


def _tc_body(xs_ref, p0_ref, p1_ref, wxs_ref, wagg_ref, b_ref, wdtop_ref, out_ref):
    xsb = xs_ref[0]
    aggb = p0_ref[...] + p1_ref[...]
    pre = (jnp.dot(xsb, wxs_ref[...], preferred_element_type=_f32)
           + jnp.dot(aggb, wagg_ref[0], preferred_element_type=_f32)
           + b_ref[...])
    learned = jnp.maximum(pre, 0.0)
    out_ref[0] = jnp.dot(learned, wdtop_ref[...], preferred_element_type=_f32)


_NB = 10
_RB = NP // _NB


def _tc_call(xs_pad, p0, p1, wxs, wagg, bfe2, wdtop):
    return pl.pallas_call(
        _tc_body,
        grid=(T, _NB),
        in_specs=[
            pl.BlockSpec((1, _RB, 16), lambda t, nb: (t, nb, 0)),
            pl.BlockSpec((_RB, 32), lambda t, nb: (nb, 0)),
            pl.BlockSpec((_RB, 32), lambda t, nb: (nb, 0)),
            pl.BlockSpec((16, 128), lambda t, nb: (0, 0)),
            pl.BlockSpec((1, 32, 128), lambda t, nb: (t, 0, 0)),
            pl.BlockSpec((1, 128), lambda t, nb: (0, 0)),
            pl.BlockSpec((128, 16), lambda t, nb: (0, 0)),
        ],
        out_specs=pl.BlockSpec((1, _RB, 16), lambda t, nb: (t, nb, 0)),
        out_shape=jax.ShapeDtypeStruct((T, NP, 16), _f32),
    )(xs_pad, p0, p1, wxs, wagg, bfe2, wdtop)


def kernel(x, edge_index, Wfe_self, Wfe_nbr, bfe, Wd_self, Wd_nbr, bd,
           Wi_self, Wi_nbr, bi):
    f32 = _f32
    s = jnp.transpose(x[:, :, :SF], (1, 0, 2)).reshape(N, T * SF)
    s_pad = jnp.zeros((NP, 32), f32).at[:N, :T * SF].set(s)
    z32 = jnp.zeros((NP, 32), f32)

    src = edge_index[0]
    dst = edge_index[1]
    pad1 = jnp.full((E1 - E,), PADROW, _i32)
    src1 = jnp.concatenate([src, pad1]).reshape(W1, E1C, CH)
    dst1 = jnp.concatenate([dst, pad1]).reshape(W1, E1C, CH)

    xs_pad = jnp.zeros((T, NP, 16), f32).at[:, :N, :SF].set(x[:, :, :SF])
    wxs = jnp.zeros((16, H), f32).at[:SF].set(Wfe_self)
    wagg = jnp.zeros((T, 32, H), f32)
    for t in range(T):
        wagg = wagg.at[t, t * SF:(t + 1) * SF].set(Wfe_nbr)
    bfe2 = bfe.reshape(1, H)
    wdtop = jnp.concatenate(
        [Wd_self[:H], Wd_nbr[:H], jnp.zeros((H, 8), f32)], axis=1)

    partials = _get_stage1()(s_pad, z32, src1, dst1)
    ab = _tc_call(xs_pad, partials[0], partials[1], wxs, wagg, bfe2, wdtop)
    spmm = _get_stage1()

    def segsum4(h4):
        hp = jnp.zeros((NP, 32), f32).at[:N, :4].set(h4)
        parts = spmm(hp, z32, src1, dst1)
        return (parts[0] + parts[1])[:N, :4]

    abn = ab[:, :N, :]
    dyn = x[0, :, SF:SF + DF]
    preds = []
    for t in range(T):
        u = abn[t, :, 4:8] + dyn @ Wd_nbr[H:]
        agg1 = segsum4(u)
        t_dot = abn[t, :, 0:4] + dyn @ Wd_self[H:] + agg1 + bd
        v = t_dot @ Wi_nbr
        agg2 = segsum4(v)
        dyn = dyn + (t_dot @ Wi_self + agg2 + bi)
        preds.append(dyn)
    return jnp.stack(preds)
